# baseline (speedup 1.0000x reference)
import functools

import jax
import jax.numpy as jnp
from jax import lax
from jax.experimental import pallas as pl
from jax.experimental.pallas import tpu as pltpu
from jax.experimental.pallas import tpu_sc as plsc

B = 8
M = 16384
T = B * M
CSEED = 128
FVOTE = 128
P = 256
K = 32
AGG = 256
NCLS = 10
RADIUS = 0.3
EPS = 1e-5
G = B * P * K


_VT = 4096


def _vote1_body(x_ref, wo_ref, bo_ref, wf_ref, bf_ref, yo_ref, yf_ref, st_ref):
    i = pl.program_id(0)
    x = x_ref[...]
    yo = jnp.dot(x, wo_ref[...], preferred_element_type=jnp.float32) + bo_ref[...]
    yf = jnp.dot(x, wf_ref[...], preferred_element_type=jnp.float32) + bf_ref[...]
    yo_ref[...] = yo
    yf_ref[...] = yf
    z = jnp.zeros((4, CSEED), jnp.float32)
    s = jnp.concatenate([
        jnp.sum(yo, axis=0, keepdims=True),
        jnp.sum(yo * yo, axis=0, keepdims=True),
        jnp.sum(yf, axis=0, keepdims=True),
        jnp.sum(yf * yf, axis=0, keepdims=True),
        z,
    ], axis=0)

    @pl.when(i == 0)
    def _():
        st_ref[...] = s

    @pl.when(i > 0)
    def _():
        st_ref[...] = st_ref[...] + s


def _vote_pass1(x, wo_t, bo, wf_t, bf):
    nblk = T // _VT
    return pl.pallas_call(
        _vote1_body,
        grid=(nblk,),
        in_specs=[
            pl.BlockSpec((_VT, CSEED), lambda i: (i, 0)),
            pl.BlockSpec((CSEED, CSEED), lambda i: (0, 0)),
            pl.BlockSpec((1, CSEED), lambda i: (0, 0)),
            pl.BlockSpec((CSEED, CSEED), lambda i: (0, 0)),
            pl.BlockSpec((1, CSEED), lambda i: (0, 0)),
        ],
        out_specs=[
            pl.BlockSpec((_VT, CSEED), lambda i: (i, 0)),
            pl.BlockSpec((_VT, CSEED), lambda i: (i, 0)),
            pl.BlockSpec((8, CSEED), lambda i: (0, 0)),
        ],
        out_shape=[
            jax.ShapeDtypeStruct((T, CSEED), jnp.float32),
            jax.ShapeDtypeStruct((T, CSEED), jnp.float32),
            jax.ShapeDtypeStruct((8, CSEED), jnp.float32),
        ],
    )(x, wo_t, bo, wf_t, bf)


def _vote2_body(yo_ref, yf_ref, st_ref, sxyz_ref, go_ref, beo_ref, gf_ref,
                bef_ref, wo2_ref, bo2_ref, wf2_ref, bf2_ref, votes_ref, vf_ref):
    st = st_ref[...]
    n = jnp.float32(T)
    mo = st[0:1, :] / n
    vo = st[1:2, :] / n - mo * mo
    mf = st[2:3, :] / n
    vf_v = st[3:4, :] / n - mf * mf

    yo = yo_ref[...]
    xo = jax.nn.relu(go_ref[...] * (yo - mo) / jnp.sqrt(vo + EPS) + beo_ref[...])
    yf = yf_ref[...]
    xf = jax.nn.relu(gf_ref[...] * (yf - mf) / jnp.sqrt(vf_v + EPS) + bef_ref[...])

    off = jnp.dot(xo, wo2_ref[...], preferred_element_type=jnp.float32)
    votes_ref[...] = sxyz_ref[...] + (off[:, :2] + bo2_ref[0:1, :2])

    vf_ref[...] = jnp.dot(xf, wf2_ref[...], preferred_element_type=jnp.float32) + bf2_ref[...]


def _vote_pass2(yo, yf, st, sxyz, go, beo, gf, bef, wo2, bo2, wf2_t, bf2):
    nblk = T // _VT
    return pl.pallas_call(
        _vote2_body,
        grid=(nblk,),
        in_specs=[
            pl.BlockSpec((_VT, CSEED), lambda i: (i, 0)),
            pl.BlockSpec((_VT, CSEED), lambda i: (i, 0)),
            pl.BlockSpec((8, CSEED), lambda i: (0, 0)),
            pl.BlockSpec((_VT, 2), lambda i: (i, 0)),
            pl.BlockSpec((1, CSEED), lambda i: (0, 0)),
            pl.BlockSpec((1, CSEED), lambda i: (0, 0)),
            pl.BlockSpec((1, CSEED), lambda i: (0, 0)),
            pl.BlockSpec((1, CSEED), lambda i: (0, 0)),
            pl.BlockSpec((CSEED, CSEED), lambda i: (0, 0)),
            pl.BlockSpec((1, CSEED), lambda i: (0, 0)),
            pl.BlockSpec((CSEED, FVOTE), lambda i: (0, 0)),
            pl.BlockSpec((1, FVOTE), lambda i: (0, 0)),
        ],
        out_specs=[
            pl.BlockSpec((_VT, 2), lambda i: (i, 0)),
            pl.BlockSpec((_VT, FVOTE), lambda i: (i, 0)),
        ],
        out_shape=[
            jax.ShapeDtypeStruct((T, 2), jnp.float32),
            jax.ShapeDtypeStruct((T, FVOTE), jnp.float32),
        ],
    )(yo, yf, st, sxyz, go, beo, gf, bef, wo2, bo2, wf2_t, bf2)


def _fps_body(vx_ref, vy_ref, nx_ref, ny_ref, mind_ref, far_ref):
    i = pl.program_id(0)

    @pl.when(i == 0)
    def _():
        mind_ref[...] = jnp.full((B, M), 1e10, jnp.float32)
        far_ref[...] = jnp.zeros((8, 128), jnp.int32)

    far = far_ref[:, 0:1]
    lanes = lax.broadcasted_iota(jnp.int32, (B, M), 1)
    selm = lanes == far
    vx = vx_ref[...]
    vy = vy_ref[...]
    cx = jnp.sum(jnp.where(selm, vx, 0.0), axis=1, keepdims=True)
    cy = jnp.sum(jnp.where(selm, vy, 0.0), axis=1, keepdims=True)
    planes = lax.broadcasted_iota(jnp.int32, (B, P), 1)
    colm = planes == i

    @pl.when(i == 0)
    def _():
        nx_ref[...] = jnp.zeros((B, P), jnp.float32)
        ny_ref[...] = jnp.zeros((B, P), jnp.float32)

    nx_ref[...] = jnp.where(colm, cx, nx_ref[...])
    ny_ref[...] = jnp.where(colm, cy, ny_ref[...])

    dx = vx - cx
    dy = vy - cy
    d = dx * dx + dy * dy
    mind = jnp.minimum(mind_ref[...], d)
    mind_ref[...] = mind
    far_new = jnp.argmax(mind, axis=1).astype(jnp.int32)
    far_ref[:, 0:1] = far_new[:, None]


def _fps(vx, vy):
    return pl.pallas_call(
        _fps_body,
        grid=(P,),
        in_specs=[
            pl.BlockSpec((B, M), lambda i: (0, 0)),
            pl.BlockSpec((B, M), lambda i: (0, 0)),
        ],
        out_specs=[
            pl.BlockSpec((B, P), lambda i: (0, 0)),
            pl.BlockSpec((B, P), lambda i: (0, 0)),
        ],
        out_shape=[
            jax.ShapeDtypeStruct((B, P), jnp.float32),
            jax.ShapeDtypeStruct((B, P), jnp.float32),
        ],
        scratch_shapes=[
            pltpu.VMEM((B, M), jnp.float32),
            pltpu.VMEM((8, 128), jnp.int32),
        ],
    )(vx, vy)


_PR = 64
_NC = 128
_CW = 128

_INF = float('inf')


def _ballq_body(vx_ref, vy_ref, nxy_ref, out_ref, d_ref, cmin_ref, cpos_ref):
    b = pl.program_id(0)
    r = pl.program_id(1)
    row0 = b * P + r * _PR
    nx = nxy_ref[pl.ds(row0, _PR), 0:1].reshape(_PR, 1, 1)
    ny = nxy_ref[pl.ds(row0, _PR), 1:2].reshape(_PR, 1, 1)
    vx3 = vx_ref[pl.ds(b, 1), :, :]
    vy3 = vy_ref[pl.ds(b, 1), :, :]
    dx = nx - vx3
    dy = ny - vy3
    d3 = dx * dx + dy * dy
    d_ref[...] = d3

    lanef3 = lax.broadcasted_iota(jnp.int32, (_PR, _NC, _CW), 2).astype(jnp.float32)
    lane2 = lax.broadcasted_iota(jnp.int32, (_PR, _NC), 1)

    m1 = jnp.min(d3, axis=2)
    p1 = jnp.min(jnp.where(d3 == m1[:, :, None], lanef3, _INF), axis=2)
    d3m = jnp.where(lanef3 == p1[:, :, None], _INF, d3)
    m2 = jnp.min(d3m, axis=2)
    p2 = jnp.min(jnp.where(d3m == m2[:, :, None], lanef3, _INF), axis=2)

    cmin_ref[...] = m1
    cpos_ref[...] = p1
    cnt = jnp.zeros((_PR, _NC), jnp.int32)

    cols = []
    for k in range(K):
        cmin = cmin_ref[...]
        m = jnp.min(cmin, axis=1)
        c = jnp.argmin(cmin, axis=1).astype(jnp.int32)
        onehot = lane2 == c[:, None]
        pos = jnp.sum(jnp.where(onehot, cpos_ref[...], 0.0), axis=1)
        j = c * _CW + pos.astype(jnp.int32)
        jeff = jnp.where(jnp.sqrt(m) > RADIUS, 0, j) + b * M
        cols.append(jeff[:, None])

        cnt = cnt + onehot.astype(jnp.int32)
        upd1 = onehot & (cnt == 1)
        pend = onehot & (cnt >= 2)
        cmin = jnp.where(upd1, m2, cmin)
        cmin = jnp.where(pend, _INF, cmin)
        cmin_ref[...] = cmin
        cpos_ref[...] = jnp.where(upd1, p2, cpos_ref[...])

        anyp = jnp.any(pend)

        @pl.when(anyp)
        def _(pend=pend, m=m, pos=pos):
            pen3 = jnp.where(pend, 0.0, _INF)[:, :, None]
            rows = jnp.min(d_ref[...] + pen3, axis=1)
            lanec = lax.broadcasted_iota(jnp.int32, (_PR, _CW), 1).astype(jnp.float32)
            keep = (rows > m[:, None]) | ((rows == m[:, None]) & (lanec > pos[:, None]))
            rk = jnp.where(keep, rows, _INF)
            nm = jnp.min(rk, axis=1)
            npos = jnp.min(jnp.where(rk == nm[:, None], lanec, _INF), axis=1)
            cmin_ref[...] = jnp.where(pend, nm[:, None], cmin_ref[...])
            cpos_ref[...] = jnp.where(pend, npos[:, None], cpos_ref[...])

    out_ref[0] = jnp.concatenate(cols, axis=1)


def _ballq(vx3, vy3, nxy):
    return pl.pallas_call(
        _ballq_body,
        grid=(B, P // _PR),
        in_specs=[
            pl.BlockSpec((B, _NC, _CW), lambda b, r: (0, 0, 0)),
            pl.BlockSpec((B, _NC, _CW), lambda b, r: (0, 0, 0)),
            pl.BlockSpec((B * P, 2), lambda b, r: (0, 0)),
        ],
        out_specs=pl.BlockSpec((1, _PR, K), lambda b, r: (b, r, 0)),
        out_shape=jax.ShapeDtypeStruct((B, P, K), jnp.int32),
        scratch_shapes=[
            pltpu.VMEM((_PR, _NC, _CW), jnp.float32),
            pltpu.VMEM((_PR, _NC), jnp.float32),
            pltpu.VMEM((_PR, _NC), jnp.float32),
        ],
    )(vx3, vy3, nxy)


_NW = 32
_RPW = G // _NW
_NCH = _RPW // 128


def _scgather_body(gidx_hbm, feats_hbm, vx_hbm, vy_hbm, nx_hbm, ny_hbm,
                   fout_hbm, xout_hbm, yout_hbm,
                   idx_v, rows_v, vx_v, vy_v, nx_v, ny_v, xg_v, yg_v, sem):
    wid = lax.axis_index("s") * 2 + lax.axis_index("c")
    base = wid * _RPW
    b = base // (P * K)
    pltpu.sync_copy(vx_hbm.at[pl.ds(b * M, M)], vx_v)
    pltpu.sync_copy(vy_hbm.at[pl.ds(b * M, M)], vy_v)
    pltpu.sync_copy(nx_hbm.at[pl.ds(b * P, P)], nx_v)
    pltpu.sync_copy(ny_hbm.at[pl.ds(b * P, P)], ny_v)
    for c in range(_NCH):
        row0 = base + c * 128
        pltpu.sync_copy(gidx_hbm.at[pl.ds(row0, 128)], idx_v)
        pltpu.async_copy(feats_hbm.at[idx_v], rows_v, sem).wait()
        pltpu.sync_copy(rows_v, fout_hbm.at[pl.ds(row0, 128)])
        for k in range(8):
            off = c * 128 + k * 16
            jv = idx_v[pl.ds(k * 16, 16)] - b * M
            rv = (base + off) + lax.iota(jnp.int32, 16)
            pv = (rv & (P * K - 1)) >> 5
            xc = plsc.load_gather(nx_v, [pv])
            yc = plsc.load_gather(ny_v, [pv])
            xv = plsc.load_gather(vx_v, [jv])
            yv = plsc.load_gather(vy_v, [jv])
            xg_v[pl.ds(off, 16)] = xv - xc
            yg_v[pl.ds(off, 16)] = yv - yc
    pltpu.sync_copy(xg_v, xout_hbm.at[pl.ds(base, _RPW)])
    pltpu.sync_copy(yg_v, yout_hbm.at[pl.ds(base, _RPW)])


def _scgather(gidx, feats, vx_flat, vy_flat, nx_flat, ny_flat):
    mesh = plsc.VectorSubcoreMesh(core_axis_name="c", subcore_axis_name="s")
    fn = functools.partial(
        pl.kernel,
        mesh=mesh,
        compiler_params=pltpu.CompilerParams(needs_layout_passes=False),
        out_type=[
            jax.ShapeDtypeStruct((G, FVOTE), jnp.float32),
            jax.ShapeDtypeStruct((G,), jnp.float32),
            jax.ShapeDtypeStruct((G,), jnp.float32),
        ],
        scratch_types=[
            pltpu.VMEM((128,), jnp.int32),
            pltpu.VMEM((128, FVOTE), jnp.float32),
            pltpu.VMEM((M,), jnp.float32),
            pltpu.VMEM((M,), jnp.float32),
            pltpu.VMEM((P,), jnp.float32),
            pltpu.VMEM((P,), jnp.float32),
            pltpu.VMEM((_RPW,), jnp.float32),
            pltpu.VMEM((_RPW,), jnp.float32),
            pltpu.SemaphoreType.DMA,
        ],
    )(_scgather_body)
    return fn(gidx, feats, vx_flat, vy_flat, nx_flat, ny_flat)


_GT = 2048


def _agg1_body(fg_ref, xgn_ref, ygn_ref, w1f_ref, c01_ref, b1_ref, y_ref, st_ref):
    i = pl.program_id(0)
    y = jnp.dot(fg_ref[...], w1f_ref[...], preferred_element_type=jnp.float32)
    y = y + xgn_ref[...] * c01_ref[0:1, :] + ygn_ref[...] * c01_ref[1:2, :] + b1_ref[...]
    y_ref[...] = y
    z = jnp.zeros((6, AGG), jnp.float32)
    s = jnp.concatenate([
        jnp.sum(y, axis=0, keepdims=True),
        jnp.sum(y * y, axis=0, keepdims=True),
        z,
    ], axis=0)

    @pl.when(i == 0)
    def _():
        st_ref[...] = s

    @pl.when(i > 0)
    def _():
        st_ref[...] = st_ref[...] + s


def _agg1(fg, xgn, ygn, w1f_t, c01, b1):
    nblk = G // _GT
    return pl.pallas_call(
        _agg1_body,
        grid=(nblk,),
        in_specs=[
            pl.BlockSpec((_GT, FVOTE), lambda i: (i, 0)),
            pl.BlockSpec((_GT, 1), lambda i: (i, 0)),
            pl.BlockSpec((_GT, 1), lambda i: (i, 0)),
            pl.BlockSpec((FVOTE, AGG), lambda i: (0, 0)),
            pl.BlockSpec((8, AGG), lambda i: (0, 0)),
            pl.BlockSpec((1, AGG), lambda i: (0, 0)),
        ],
        out_specs=[
            pl.BlockSpec((_GT, AGG), lambda i: (i, 0)),
            pl.BlockSpec((8, AGG), lambda i: (0, 0)),
        ],
        out_shape=[
            jax.ShapeDtypeStruct((G, AGG), jnp.float32),
            jax.ShapeDtypeStruct((8, AGG), jnp.float32),
        ],
    )(fg, xgn, ygn, w1f_t, c01, b1)


def _agg2_body(y1_ref, st_ref, g_ref, be_ref, w2_ref, b2_ref, y_ref, st2_ref):
    i = pl.program_id(0)
    st = st_ref[...]
    n = jnp.float32(G)
    m = st[0:1, :] / n
    v = st[1:2, :] / n - m * m
    h = jax.nn.relu(g_ref[...] * (y1_ref[...] - m) / jnp.sqrt(v + EPS) + be_ref[...])
    y = jnp.dot(h, w2_ref[...], preferred_element_type=jnp.float32) + b2_ref[...]
    y_ref[...] = y
    z = jnp.zeros((6, AGG), jnp.float32)
    s = jnp.concatenate([
        jnp.sum(y, axis=0, keepdims=True),
        jnp.sum(y * y, axis=0, keepdims=True),
        z,
    ], axis=0)

    @pl.when(i == 0)
    def _():
        st2_ref[...] = s

    @pl.when(i > 0)
    def _():
        st2_ref[...] = st2_ref[...] + s


def _agg2(y1, st1, g, be, w2_t, b2):
    nblk = G // _GT
    return pl.pallas_call(
        _agg2_body,
        grid=(nblk,),
        in_specs=[
            pl.BlockSpec((_GT, AGG), lambda i: (i, 0)),
            pl.BlockSpec((8, AGG), lambda i: (0, 0)),
            pl.BlockSpec((1, AGG), lambda i: (0, 0)),
            pl.BlockSpec((1, AGG), lambda i: (0, 0)),
            pl.BlockSpec((AGG, AGG), lambda i: (0, 0)),
            pl.BlockSpec((1, AGG), lambda i: (0, 0)),
        ],
        out_specs=[
            pl.BlockSpec((_GT, AGG), lambda i: (i, 0)),
            pl.BlockSpec((8, AGG), lambda i: (0, 0)),
        ],
        out_shape=[
            jax.ShapeDtypeStruct((G, AGG), jnp.float32),
            jax.ShapeDtypeStruct((8, AGG), jnp.float32),
        ],
    )(y1, st1, g, be, w2_t, b2)


def _agg3_body(y2_ref, st_ref, g_ref, be_ref, pool_ref):
    st = st_ref[...]
    n = jnp.float32(G)
    m = st[0:1, :] / n
    v = st[1:2, :] / n - m * m
    h = jax.nn.relu(g_ref[...] * (y2_ref[...] - m) / jnp.sqrt(v + EPS) + be_ref[...])
    h3 = h.reshape(_GT // K, K, AGG)
    pool_ref[...] = jnp.max(h3, axis=1)


def _agg3(y2, st2, g, be):
    nblk = G // _GT
    return pl.pallas_call(
        _agg3_body,
        grid=(nblk,),
        in_specs=[
            pl.BlockSpec((_GT, AGG), lambda i: (i, 0)),
            pl.BlockSpec((8, AGG), lambda i: (0, 0)),
            pl.BlockSpec((1, AGG), lambda i: (0, 0)),
            pl.BlockSpec((1, AGG), lambda i: (0, 0)),
        ],
        out_specs=pl.BlockSpec((_GT // K, AGG), lambda i: (i, 0)),
        out_shape=jax.ShapeDtypeStruct((B * P, AGG), jnp.float32),
    )(y2, st2, g, be)


_HT = B * P


def _head_body(x_ref, w1_ref, b1_ref, g1_ref, be1_ref, w2_ref, b2_ref, g2_ref,
               be2_ref, ow_ref, ob_ref, cw_ref, cb_ref, obj_ref, cls_ref):
    n = jnp.float32(_HT)
    y1 = jnp.dot(x_ref[...], w1_ref[...], preferred_element_type=jnp.float32) + b1_ref[...]
    m1 = jnp.sum(y1, axis=0, keepdims=True) / n
    v1 = jnp.sum(y1 * y1, axis=0, keepdims=True) / n - m1 * m1
    h1 = jax.nn.relu(g1_ref[...] * (y1 - m1) / jnp.sqrt(v1 + EPS) + be1_ref[...])
    y2 = jnp.dot(h1, w2_ref[...], preferred_element_type=jnp.float32) + b2_ref[...]
    m2 = jnp.sum(y2, axis=0, keepdims=True) / n
    v2 = jnp.sum(y2 * y2, axis=0, keepdims=True) / n - m2 * m2
    h2 = jax.nn.relu(g2_ref[...] * (y2 - m2) / jnp.sqrt(v2 + EPS) + be2_ref[...])
    obj_ref[...] = jnp.sum(h2 * ow_ref[...], axis=1, keepdims=True) + ob_ref[0:1, 0:1]
    cls_ref[...] = jnp.dot(h2, cw_ref[...], preferred_element_type=jnp.float32) + cb_ref[...]


def _head(x, w1_t, b1, g1, be1, w2_t, b2, g2, be2, ow, ob, cw_t, cb):
    return pl.pallas_call(
        _head_body,
        grid=(1,),
        in_specs=[
            pl.BlockSpec((_HT, AGG), lambda i: (0, 0)),
            pl.BlockSpec((AGG, AGG), lambda i: (0, 0)),
            pl.BlockSpec((1, AGG), lambda i: (0, 0)),
            pl.BlockSpec((1, AGG), lambda i: (0, 0)),
            pl.BlockSpec((1, AGG), lambda i: (0, 0)),
            pl.BlockSpec((AGG, AGG // 2), lambda i: (0, 0)),
            pl.BlockSpec((1, AGG // 2), lambda i: (0, 0)),
            pl.BlockSpec((1, AGG // 2), lambda i: (0, 0)),
            pl.BlockSpec((1, AGG // 2), lambda i: (0, 0)),
            pl.BlockSpec((1, AGG // 2), lambda i: (0, 0)),
            pl.BlockSpec((8, 8), lambda i: (0, 0)),
            pl.BlockSpec((AGG // 2, 16), lambda i: (0, 0)),
            pl.BlockSpec((1, 16), lambda i: (0, 0)),
        ],
        out_specs=[
            pl.BlockSpec((_HT, 1), lambda i: (0, 0)),
            pl.BlockSpec((_HT, 16), lambda i: (0, 0)),
        ],
        out_shape=[
            jax.ShapeDtypeStruct((_HT, 1), jnp.float32),
            jax.ShapeDtypeStruct((_HT, 16), jnp.float32),
        ],
    )(x, w1_t, b1, g1, be1, w2_t, b2, g2, be2, ow, ob, cw_t, cb)


def kernel(seed_xyz, seed_features, v_ow1, v_ob1, v_og1, v_obt1, v_ow2, v_ob2,
           v_fw1, v_fb1, v_fg1, v_fbt1, v_fw2, v_fb2, a_w1, a_b1, a_g1, a_bt1,
           a_w2, a_b2, a_g2, a_bt2, h_w1, h_b1, h_g1, h_bt1, h_w2, h_b2, h_g2,
           h_bt2, h_ow, h_ob, h_cw, h_cb):
    r1 = lambda a: a.reshape(1, -1)
    x = seed_features.reshape(T, CSEED)
    sxyz = seed_xyz.reshape(T, 2)

    yo, yf, st = _vote_pass1(x, v_ow1.T, r1(v_ob1), v_fw1.T, r1(v_fb1))
    wo2 = jnp.zeros((CSEED, CSEED), jnp.float32).at[:, :2].set(v_ow2.T)
    bo2 = jnp.zeros((1, CSEED), jnp.float32).at[0, :2].set(v_ob2)
    votes, vfeat = _vote_pass2(yo, yf, st, sxyz, r1(v_og1), r1(v_obt1),
                               r1(v_fg1), r1(v_fbt1), wo2, bo2, v_fw2.T,
                               r1(v_fb2))

    vx = votes[:, 0].reshape(B, M)
    vy = votes[:, 1].reshape(B, M)
    nx, ny = _fps(vx, vy)
    nxy = jnp.stack([nx.reshape(-1), ny.reshape(-1)], axis=1)

    gidx = _ballq(vx.reshape(B, _NC, _CW), vy.reshape(B, _NC, _CW),
                  nxy).reshape(G)

    fg, xgn, ygn = _scgather(gidx, vfeat, votes[:, 0], votes[:, 1],
                             nx.reshape(-1), ny.reshape(-1))

    c01 = jnp.zeros((8, AGG), jnp.float32).at[:2].set(a_w1[:, :2].T)
    y1, st1 = _agg1(fg, xgn.reshape(G, 1), ygn.reshape(G, 1), a_w1[:, 2:].T,
                    c01, r1(a_b1))
    y2, st2 = _agg2(y1, st1, r1(a_g1), r1(a_bt1), a_w2.T, r1(a_b2))
    pooled = _agg3(y2, st2, r1(a_g2), r1(a_bt2))

    ow = jnp.zeros((8, 8), jnp.float32).at[0, 0].set(h_ob[0])
    cw = jnp.zeros((16, AGG // 2), jnp.float32).at[:NCLS].set(h_cw)
    cb = jnp.zeros((1, 16), jnp.float32).at[0, :NCLS].set(h_cb)
    obj2, cls2 = _head(pooled, h_w1.T, r1(h_b1), r1(h_g1), r1(h_bt1), h_w2.T,
                       r1(h_b2), r1(h_g2), r1(h_bt2), r1(h_ow), ow, cw.T, cb)

    obj = obj2[:, 0].reshape(B, P)
    cls = cls2[:, :NCLS].reshape(B, P, NCLS)
    new_xyz = jnp.stack([nx, ny], axis=-1)
    return (obj, cls, new_xyz)


# speedup vs baseline: 2.1177x; 2.1177x over previous
import functools

import jax
import jax.numpy as jnp
from jax import lax
from jax.experimental import pallas as pl
from jax.experimental.pallas import tpu as pltpu
from jax.experimental.pallas import tpu_sc as plsc

B = 8
M = 16384
T = B * M
CSEED = 128
FVOTE = 128
P = 256
K = 32
AGG = 256
NCLS = 10
RADIUS = 0.3
EPS = 1e-5
G = B * P * K


_VT = 4096


def _vote1_body(x_ref, wo_ref, bo_ref, wf_ref, bf_ref, yo_ref, yf_ref, st_ref):
    i = pl.program_id(0)
    x = x_ref[...]
    yo = jnp.dot(x, wo_ref[...], preferred_element_type=jnp.float32) + bo_ref[...]
    yf = jnp.dot(x, wf_ref[...], preferred_element_type=jnp.float32) + bf_ref[...]
    yo_ref[...] = yo
    yf_ref[...] = yf
    z = jnp.zeros((4, CSEED), jnp.float32)
    s = jnp.concatenate([
        jnp.sum(yo, axis=0, keepdims=True),
        jnp.sum(yo * yo, axis=0, keepdims=True),
        jnp.sum(yf, axis=0, keepdims=True),
        jnp.sum(yf * yf, axis=0, keepdims=True),
        z,
    ], axis=0)

    @pl.when(i == 0)
    def _():
        st_ref[...] = s

    @pl.when(i > 0)
    def _():
        st_ref[...] = st_ref[...] + s


def _vote_pass1(x, wo_t, bo, wf_t, bf):
    nblk = T // _VT
    return pl.pallas_call(
        _vote1_body,
        grid=(nblk,),
        in_specs=[
            pl.BlockSpec((_VT, CSEED), lambda i: (i, 0)),
            pl.BlockSpec((CSEED, CSEED), lambda i: (0, 0)),
            pl.BlockSpec((1, CSEED), lambda i: (0, 0)),
            pl.BlockSpec((CSEED, CSEED), lambda i: (0, 0)),
            pl.BlockSpec((1, CSEED), lambda i: (0, 0)),
        ],
        out_specs=[
            pl.BlockSpec((_VT, CSEED), lambda i: (i, 0)),
            pl.BlockSpec((_VT, CSEED), lambda i: (i, 0)),
            pl.BlockSpec((8, CSEED), lambda i: (0, 0)),
        ],
        out_shape=[
            jax.ShapeDtypeStruct((T, CSEED), jnp.float32),
            jax.ShapeDtypeStruct((T, CSEED), jnp.float32),
            jax.ShapeDtypeStruct((8, CSEED), jnp.float32),
        ],
    )(x, wo_t, bo, wf_t, bf)


def _vote2_body(yo_ref, yf_ref, st_ref, sxyz_ref, go_ref, beo_ref, gf_ref,
                bef_ref, wo2_ref, bo2_ref, wf2_ref, bf2_ref, votes_ref, vf_ref):
    st = st_ref[...]
    n = jnp.float32(T)
    mo = st[0:1, :] / n
    vo = st[1:2, :] / n - mo * mo
    mf = st[2:3, :] / n
    vf_v = st[3:4, :] / n - mf * mf

    yo = yo_ref[...]
    xo = jax.nn.relu(go_ref[...] * (yo - mo) / jnp.sqrt(vo + EPS) + beo_ref[...])
    yf = yf_ref[...]
    xf = jax.nn.relu(gf_ref[...] * (yf - mf) / jnp.sqrt(vf_v + EPS) + bef_ref[...])

    off = jnp.dot(xo, wo2_ref[...], preferred_element_type=jnp.float32)
    votes_ref[...] = sxyz_ref[...] + (off[:, :2] + bo2_ref[0:1, :2])

    vf_ref[...] = jnp.dot(xf, wf2_ref[...], preferred_element_type=jnp.float32) + bf2_ref[...]


def _vote_pass2(yo, yf, st, sxyz, go, beo, gf, bef, wo2, bo2, wf2_t, bf2):
    nblk = T // _VT
    return pl.pallas_call(
        _vote2_body,
        grid=(nblk,),
        in_specs=[
            pl.BlockSpec((_VT, CSEED), lambda i: (i, 0)),
            pl.BlockSpec((_VT, CSEED), lambda i: (i, 0)),
            pl.BlockSpec((8, CSEED), lambda i: (0, 0)),
            pl.BlockSpec((_VT, 2), lambda i: (i, 0)),
            pl.BlockSpec((1, CSEED), lambda i: (0, 0)),
            pl.BlockSpec((1, CSEED), lambda i: (0, 0)),
            pl.BlockSpec((1, CSEED), lambda i: (0, 0)),
            pl.BlockSpec((1, CSEED), lambda i: (0, 0)),
            pl.BlockSpec((CSEED, CSEED), lambda i: (0, 0)),
            pl.BlockSpec((1, CSEED), lambda i: (0, 0)),
            pl.BlockSpec((CSEED, FVOTE), lambda i: (0, 0)),
            pl.BlockSpec((1, FVOTE), lambda i: (0, 0)),
        ],
        out_specs=[
            pl.BlockSpec((_VT, 2), lambda i: (i, 0)),
            pl.BlockSpec((_VT, FVOTE), lambda i: (i, 0)),
        ],
        out_shape=[
            jax.ShapeDtypeStruct((T, 2), jnp.float32),
            jax.ShapeDtypeStruct((T, FVOTE), jnp.float32),
        ],
    )(yo, yf, st, sxyz, go, beo, gf, bef, wo2, bo2, wf2_t, bf2)


def _fps_body(vx_ref, vy_ref, nx_ref, ny_ref, mind_ref, far_ref):
    i = pl.program_id(0)

    @pl.when(i == 0)
    def _():
        mind_ref[...] = jnp.full((B, M), 1e10, jnp.float32)
        far_ref[...] = jnp.zeros((8, 128), jnp.int32)

    far = far_ref[:, 0:1]
    lanes = lax.broadcasted_iota(jnp.int32, (B, M), 1)
    selm = lanes == far
    vx = vx_ref[...]
    vy = vy_ref[...]
    cx = jnp.sum(jnp.where(selm, vx, 0.0), axis=1, keepdims=True)
    cy = jnp.sum(jnp.where(selm, vy, 0.0), axis=1, keepdims=True)
    planes = lax.broadcasted_iota(jnp.int32, (B, P), 1)
    colm = planes == i

    @pl.when(i == 0)
    def _():
        nx_ref[...] = jnp.zeros((B, P), jnp.float32)
        ny_ref[...] = jnp.zeros((B, P), jnp.float32)

    nx_ref[...] = jnp.where(colm, cx, nx_ref[...])
    ny_ref[...] = jnp.where(colm, cy, ny_ref[...])

    dx = vx - cx
    dy = vy - cy
    d = dx * dx + dy * dy
    mind = jnp.minimum(mind_ref[...], d)
    mind_ref[...] = mind
    far_new = jnp.argmax(mind, axis=1).astype(jnp.int32)
    far_ref[:, 0:1] = far_new[:, None]


def _fps(vx, vy):
    return pl.pallas_call(
        _fps_body,
        grid=(P,),
        in_specs=[
            pl.BlockSpec((B, M), lambda i: (0, 0)),
            pl.BlockSpec((B, M), lambda i: (0, 0)),
        ],
        out_specs=[
            pl.BlockSpec((B, P), lambda i: (0, 0)),
            pl.BlockSpec((B, P), lambda i: (0, 0)),
        ],
        out_shape=[
            jax.ShapeDtypeStruct((B, P), jnp.float32),
            jax.ShapeDtypeStruct((B, P), jnp.float32),
        ],
        scratch_shapes=[
            pltpu.VMEM((B, M), jnp.float32),
            pltpu.VMEM((8, 128), jnp.int32),
        ],
    )(vx, vy)


_PR = 64
_NC = 512
_CW = 32

_INF = float('inf')


def _ballq_body(vx_ref, vy_ref, nxy_ref, out_ref, d_ref):
    b = pl.program_id(0)
    r = pl.program_id(1)
    row0 = b * P + r * _PR
    nx = nxy_ref[pl.ds(row0, _PR), 0:1].reshape(_PR, 1, 1)
    ny = nxy_ref[pl.ds(row0, _PR), 1:2].reshape(_PR, 1, 1)
    vx3 = vx_ref[pl.ds(b, 1), :, :]
    vy3 = vy_ref[pl.ds(b, 1), :, :]
    dx = nx - vx3
    dy = ny - vy3
    d3 = dx * dx + dy * dy
    d_ref[...] = d3

    wio3 = lax.broadcasted_iota(jnp.int32, (_PR, _CW, _NC), 1).astype(jnp.float32)
    chunkf = lax.broadcasted_iota(jnp.int32, (_PR, _NC), 1).astype(jnp.float32)
    lane2 = lax.broadcasted_iota(jnp.int32, (_PR, _NC), 1)

    NL = 5
    ms, ps = [], []
    dcur = d3
    for _ in range(NL):
        mi = jnp.min(dcur, axis=1)
        pi = jnp.min(jnp.where(dcur == mi[:, None, :], wio3, _INF), axis=1)
        ms.append(mi)
        ps.append(pi)
        dcur = jnp.where(wio3 == pi[:, None, :], _INF, dcur)

    cmin = ms[0]
    cpos = ps[0]
    cnt = jnp.zeros((_PR, _NC), jnp.int32)
    bad = jnp.zeros((_PR, _NC), jnp.bool_)

    cols = []
    for k in range(K):
        m = jnp.min(cmin, axis=1, keepdims=True)
        oidx = cpos * float(_NC) + chunkf
        osel = jnp.min(jnp.where(cmin == m, oidx, _INF), axis=1)
        osel_i = osel.astype(jnp.int32)
        c = osel_i & (_NC - 1)
        mv = m[:, 0]
        jeff = jnp.where(jnp.sqrt(mv) > RADIUS, 0, osel_i) + b * M
        cols.append(jeff[:, None])

        onehot = lane2 == c[:, None]
        cnt = cnt + onehot.astype(jnp.int32)
        nxt_v = jnp.full((_PR, _NC), _INF, jnp.float32)
        nxt_p = jnp.zeros((_PR, _NC), jnp.float32)
        for lvl in range(1, NL):
            is_l = cnt == lvl
            nxt_v = jnp.where(is_l, ms[lvl], nxt_v)
            nxt_p = jnp.where(is_l, ps[lvl], nxt_p)
        cmin = jnp.where(onehot, nxt_v, cmin)
        cpos = jnp.where(onehot, nxt_p, cpos)
        bad = bad | (onehot & (cnt >= NL))

    out_ref[0] = jnp.concatenate(cols, axis=1)
    anyb = jnp.any(bad)

    @pl.when(anyb)
    def _():
        oidx3 = (wio3 * float(_NC) +
                 lax.broadcasted_iota(jnp.int32, (_PR, _CW, _NC), 2)
                 .astype(jnp.float32))
        fcols = []
        for k in range(K):
            dd = d_ref[...]
            fm2 = jnp.min(dd, axis=1)
            fm = jnp.min(fm2, axis=1)
            sel = jnp.where(dd == fm[:, None, None], oidx3, _INF)
            fo2 = jnp.min(sel, axis=1)
            fo = jnp.min(fo2, axis=1)
            d_ref[...] = jnp.where(oidx3 == fo[:, None, None], _INF, dd)
            fo_i = fo.astype(jnp.int32)
            fj = jnp.where(jnp.sqrt(fm) > RADIUS, 0, fo_i) + b * M
            fcols.append(fj[:, None])
        out_ref[0] = jnp.concatenate(fcols, axis=1)


def _ballq(vx3, vy3, nxy):
    return pl.pallas_call(
        _ballq_body,
        grid=(B, P // _PR),
        in_specs=[
            pl.BlockSpec((B, _CW, _NC), lambda b, r: (0, 0, 0)),
            pl.BlockSpec((B, _CW, _NC), lambda b, r: (0, 0, 0)),
            pl.BlockSpec((B * P, 2), lambda b, r: (0, 0)),
        ],
        out_specs=pl.BlockSpec((1, _PR, K), lambda b, r: (b, r, 0)),
        out_shape=jax.ShapeDtypeStruct((B, P, K), jnp.int32),
        scratch_shapes=[
            pltpu.VMEM((_PR, _CW, _NC), jnp.float32),
        ],
    )(vx3, vy3, nxy)


_NW = 32
_RPW = G // _NW
_NCH = _RPW // 128


def _scgather_body(gidx_hbm, feats_hbm, vx_hbm, vy_hbm, nx_hbm, ny_hbm,
                   fout_hbm, xout_hbm, yout_hbm,
                   idx_v, rows_v, vx_v, vy_v, nx_v, ny_v, xg_v, yg_v, sem):
    wid = lax.axis_index("s") * 2 + lax.axis_index("c")
    base = wid * _RPW
    b = base // (P * K)
    pltpu.sync_copy(vx_hbm.at[pl.ds(b * M, M)], vx_v)
    pltpu.sync_copy(vy_hbm.at[pl.ds(b * M, M)], vy_v)
    pltpu.sync_copy(nx_hbm.at[pl.ds(b * P, P)], nx_v)
    pltpu.sync_copy(ny_hbm.at[pl.ds(b * P, P)], ny_v)
    for c in range(_NCH):
        row0 = base + c * 128
        pltpu.sync_copy(gidx_hbm.at[pl.ds(row0, 128)], idx_v)
        pltpu.async_copy(feats_hbm.at[idx_v], rows_v, sem).wait()
        pltpu.sync_copy(rows_v, fout_hbm.at[pl.ds(row0, 128)])
        for k in range(8):
            off = c * 128 + k * 16
            jv = idx_v[pl.ds(k * 16, 16)] - b * M
            rv = (base + off) + lax.iota(jnp.int32, 16)
            pv = (rv & (P * K - 1)) >> 5
            xc = plsc.load_gather(nx_v, [pv])
            yc = plsc.load_gather(ny_v, [pv])
            xv = plsc.load_gather(vx_v, [jv])
            yv = plsc.load_gather(vy_v, [jv])
            xg_v[pl.ds(off, 16)] = xv - xc
            yg_v[pl.ds(off, 16)] = yv - yc
    pltpu.sync_copy(xg_v, xout_hbm.at[pl.ds(base, _RPW)])
    pltpu.sync_copy(yg_v, yout_hbm.at[pl.ds(base, _RPW)])


def _scgather(gidx, feats, vx_flat, vy_flat, nx_flat, ny_flat):
    mesh = plsc.VectorSubcoreMesh(core_axis_name="c", subcore_axis_name="s")
    fn = functools.partial(
        pl.kernel,
        mesh=mesh,
        compiler_params=pltpu.CompilerParams(needs_layout_passes=False),
        out_type=[
            jax.ShapeDtypeStruct((G, FVOTE), jnp.float32),
            jax.ShapeDtypeStruct((G,), jnp.float32),
            jax.ShapeDtypeStruct((G,), jnp.float32),
        ],
        scratch_types=[
            pltpu.VMEM((128,), jnp.int32),
            pltpu.VMEM((128, FVOTE), jnp.float32),
            pltpu.VMEM((M,), jnp.float32),
            pltpu.VMEM((M,), jnp.float32),
            pltpu.VMEM((P,), jnp.float32),
            pltpu.VMEM((P,), jnp.float32),
            pltpu.VMEM((_RPW,), jnp.float32),
            pltpu.VMEM((_RPW,), jnp.float32),
            pltpu.SemaphoreType.DMA,
        ],
    )(_scgather_body)
    return fn(gidx, feats, vx_flat, vy_flat, nx_flat, ny_flat)


_GT = 2048


def _agg1_body(fg_ref, xgn_ref, ygn_ref, w1f_ref, c01_ref, b1_ref, y_ref, st_ref):
    i = pl.program_id(0)
    y = jnp.dot(fg_ref[...], w1f_ref[...], preferred_element_type=jnp.float32)
    y = y + xgn_ref[...] * c01_ref[0:1, :] + ygn_ref[...] * c01_ref[1:2, :] + b1_ref[...]
    y_ref[...] = y
    z = jnp.zeros((6, AGG), jnp.float32)
    s = jnp.concatenate([
        jnp.sum(y, axis=0, keepdims=True),
        jnp.sum(y * y, axis=0, keepdims=True),
        z,
    ], axis=0)

    @pl.when(i == 0)
    def _():
        st_ref[...] = s

    @pl.when(i > 0)
    def _():
        st_ref[...] = st_ref[...] + s


def _agg1(fg, xgn, ygn, w1f_t, c01, b1):
    nblk = G // _GT
    return pl.pallas_call(
        _agg1_body,
        grid=(nblk,),
        in_specs=[
            pl.BlockSpec((_GT, FVOTE), lambda i: (i, 0)),
            pl.BlockSpec((_GT, 1), lambda i: (i, 0)),
            pl.BlockSpec((_GT, 1), lambda i: (i, 0)),
            pl.BlockSpec((FVOTE, AGG), lambda i: (0, 0)),
            pl.BlockSpec((8, AGG), lambda i: (0, 0)),
            pl.BlockSpec((1, AGG), lambda i: (0, 0)),
        ],
        out_specs=[
            pl.BlockSpec((_GT, AGG), lambda i: (i, 0)),
            pl.BlockSpec((8, AGG), lambda i: (0, 0)),
        ],
        out_shape=[
            jax.ShapeDtypeStruct((G, AGG), jnp.float32),
            jax.ShapeDtypeStruct((8, AGG), jnp.float32),
        ],
    )(fg, xgn, ygn, w1f_t, c01, b1)


def _agg2_body(y1_ref, st_ref, g_ref, be_ref, w2_ref, b2_ref, y_ref, st2_ref):
    i = pl.program_id(0)
    st = st_ref[...]
    n = jnp.float32(G)
    m = st[0:1, :] / n
    v = st[1:2, :] / n - m * m
    h = jax.nn.relu(g_ref[...] * (y1_ref[...] - m) / jnp.sqrt(v + EPS) + be_ref[...])
    y = jnp.dot(h, w2_ref[...], preferred_element_type=jnp.float32) + b2_ref[...]
    y_ref[...] = y
    z = jnp.zeros((6, AGG), jnp.float32)
    s = jnp.concatenate([
        jnp.sum(y, axis=0, keepdims=True),
        jnp.sum(y * y, axis=0, keepdims=True),
        z,
    ], axis=0)

    @pl.when(i == 0)
    def _():
        st2_ref[...] = s

    @pl.when(i > 0)
    def _():
        st2_ref[...] = st2_ref[...] + s


def _agg2(y1, st1, g, be, w2_t, b2):
    nblk = G // _GT
    return pl.pallas_call(
        _agg2_body,
        grid=(nblk,),
        in_specs=[
            pl.BlockSpec((_GT, AGG), lambda i: (i, 0)),
            pl.BlockSpec((8, AGG), lambda i: (0, 0)),
            pl.BlockSpec((1, AGG), lambda i: (0, 0)),
            pl.BlockSpec((1, AGG), lambda i: (0, 0)),
            pl.BlockSpec((AGG, AGG), lambda i: (0, 0)),
            pl.BlockSpec((1, AGG), lambda i: (0, 0)),
        ],
        out_specs=[
            pl.BlockSpec((_GT, AGG), lambda i: (i, 0)),
            pl.BlockSpec((8, AGG), lambda i: (0, 0)),
        ],
        out_shape=[
            jax.ShapeDtypeStruct((G, AGG), jnp.float32),
            jax.ShapeDtypeStruct((8, AGG), jnp.float32),
        ],
    )(y1, st1, g, be, w2_t, b2)


def _agg3_body(y2_ref, st_ref, g_ref, be_ref, pool_ref):
    st = st_ref[...]
    n = jnp.float32(G)
    m = st[0:1, :] / n
    v = st[1:2, :] / n - m * m
    h = jax.nn.relu(g_ref[...] * (y2_ref[...] - m) / jnp.sqrt(v + EPS) + be_ref[...])
    h3 = h.reshape(_GT // K, K, AGG)
    pool_ref[...] = jnp.max(h3, axis=1)


def _agg3(y2, st2, g, be):
    nblk = G // _GT
    return pl.pallas_call(
        _agg3_body,
        grid=(nblk,),
        in_specs=[
            pl.BlockSpec((_GT, AGG), lambda i: (i, 0)),
            pl.BlockSpec((8, AGG), lambda i: (0, 0)),
            pl.BlockSpec((1, AGG), lambda i: (0, 0)),
            pl.BlockSpec((1, AGG), lambda i: (0, 0)),
        ],
        out_specs=pl.BlockSpec((_GT // K, AGG), lambda i: (i, 0)),
        out_shape=jax.ShapeDtypeStruct((B * P, AGG), jnp.float32),
    )(y2, st2, g, be)


_HT = B * P


def _head_body(x_ref, w1_ref, b1_ref, g1_ref, be1_ref, w2_ref, b2_ref, g2_ref,
               be2_ref, ow_ref, ob_ref, cw_ref, cb_ref, obj_ref, cls_ref):
    n = jnp.float32(_HT)
    y1 = jnp.dot(x_ref[...], w1_ref[...], preferred_element_type=jnp.float32) + b1_ref[...]
    m1 = jnp.sum(y1, axis=0, keepdims=True) / n
    v1 = jnp.sum(y1 * y1, axis=0, keepdims=True) / n - m1 * m1
    h1 = jax.nn.relu(g1_ref[...] * (y1 - m1) / jnp.sqrt(v1 + EPS) + be1_ref[...])
    y2 = jnp.dot(h1, w2_ref[...], preferred_element_type=jnp.float32) + b2_ref[...]
    m2 = jnp.sum(y2, axis=0, keepdims=True) / n
    v2 = jnp.sum(y2 * y2, axis=0, keepdims=True) / n - m2 * m2
    h2 = jax.nn.relu(g2_ref[...] * (y2 - m2) / jnp.sqrt(v2 + EPS) + be2_ref[...])
    obj_ref[...] = jnp.sum(h2 * ow_ref[...], axis=1, keepdims=True) + ob_ref[0:1, 0:1]
    cls_ref[...] = jnp.dot(h2, cw_ref[...], preferred_element_type=jnp.float32) + cb_ref[...]


def _head(x, w1_t, b1, g1, be1, w2_t, b2, g2, be2, ow, ob, cw_t, cb):
    return pl.pallas_call(
        _head_body,
        grid=(1,),
        in_specs=[
            pl.BlockSpec((_HT, AGG), lambda i: (0, 0)),
            pl.BlockSpec((AGG, AGG), lambda i: (0, 0)),
            pl.BlockSpec((1, AGG), lambda i: (0, 0)),
            pl.BlockSpec((1, AGG), lambda i: (0, 0)),
            pl.BlockSpec((1, AGG), lambda i: (0, 0)),
            pl.BlockSpec((AGG, AGG // 2), lambda i: (0, 0)),
            pl.BlockSpec((1, AGG // 2), lambda i: (0, 0)),
            pl.BlockSpec((1, AGG // 2), lambda i: (0, 0)),
            pl.BlockSpec((1, AGG // 2), lambda i: (0, 0)),
            pl.BlockSpec((1, AGG // 2), lambda i: (0, 0)),
            pl.BlockSpec((8, 8), lambda i: (0, 0)),
            pl.BlockSpec((AGG // 2, 16), lambda i: (0, 0)),
            pl.BlockSpec((1, 16), lambda i: (0, 0)),
        ],
        out_specs=[
            pl.BlockSpec((_HT, 1), lambda i: (0, 0)),
            pl.BlockSpec((_HT, 16), lambda i: (0, 0)),
        ],
        out_shape=[
            jax.ShapeDtypeStruct((_HT, 1), jnp.float32),
            jax.ShapeDtypeStruct((_HT, 16), jnp.float32),
        ],
    )(x, w1_t, b1, g1, be1, w2_t, b2, g2, be2, ow, ob, cw_t, cb)


def kernel(seed_xyz, seed_features, v_ow1, v_ob1, v_og1, v_obt1, v_ow2, v_ob2,
           v_fw1, v_fb1, v_fg1, v_fbt1, v_fw2, v_fb2, a_w1, a_b1, a_g1, a_bt1,
           a_w2, a_b2, a_g2, a_bt2, h_w1, h_b1, h_g1, h_bt1, h_w2, h_b2, h_g2,
           h_bt2, h_ow, h_ob, h_cw, h_cb):
    r1 = lambda a: a.reshape(1, -1)
    x = seed_features.reshape(T, CSEED)
    sxyz = seed_xyz.reshape(T, 2)

    yo, yf, st = _vote_pass1(x, v_ow1.T, r1(v_ob1), v_fw1.T, r1(v_fb1))
    wo2 = jnp.zeros((CSEED, CSEED), jnp.float32).at[:, :2].set(v_ow2.T)
    bo2 = jnp.zeros((1, CSEED), jnp.float32).at[0, :2].set(v_ob2)
    votes, vfeat = _vote_pass2(yo, yf, st, sxyz, r1(v_og1), r1(v_obt1),
                               r1(v_fg1), r1(v_fbt1), wo2, bo2, v_fw2.T,
                               r1(v_fb2))

    vx = votes[:, 0].reshape(B, M)
    vy = votes[:, 1].reshape(B, M)
    nx, ny = _fps(vx, vy)
    nxy = jnp.stack([nx.reshape(-1), ny.reshape(-1)], axis=1)

    gidx = _ballq(vx.reshape(B, _CW, _NC), vy.reshape(B, _CW, _NC),
                  nxy).reshape(G)

    fg, xgn, ygn = _scgather(gidx, vfeat, votes[:, 0], votes[:, 1],
                             nx.reshape(-1), ny.reshape(-1))

    c01 = jnp.zeros((8, AGG), jnp.float32).at[:2].set(a_w1[:, :2].T)
    y1, st1 = _agg1(fg, xgn.reshape(G, 1), ygn.reshape(G, 1), a_w1[:, 2:].T,
                    c01, r1(a_b1))
    y2, st2 = _agg2(y1, st1, r1(a_g1), r1(a_bt1), a_w2.T, r1(a_b2))
    pooled = _agg3(y2, st2, r1(a_g2), r1(a_bt2))

    ow = jnp.zeros((8, 8), jnp.float32).at[0, 0].set(h_ob[0])
    cw = jnp.zeros((16, AGG // 2), jnp.float32).at[:NCLS].set(h_cw)
    cb = jnp.zeros((1, 16), jnp.float32).at[0, :NCLS].set(h_cb)
    obj2, cls2 = _head(pooled, h_w1.T, r1(h_b1), r1(h_g1), r1(h_bt1), h_w2.T,
                       r1(h_b2), r1(h_g2), r1(h_bt2), r1(h_ow), ow, cw.T, cb)

    obj = obj2[:, 0].reshape(B, P)
    cls = cls2[:, :NCLS].reshape(B, P, NCLS)
    new_xyz = jnp.stack([nx, ny], axis=-1)
    return (obj, cls, new_xyz)


# speedup vs baseline: 2.2382x; 1.0569x over previous
import functools

import jax
import jax.numpy as jnp
from jax import lax
from jax.experimental import pallas as pl
from jax.experimental.pallas import tpu as pltpu
from jax.experimental.pallas import tpu_sc as plsc

B = 8
M = 16384
T = B * M
CSEED = 128
FVOTE = 128
P = 256
K = 32
AGG = 256
NCLS = 10
RADIUS = 0.3
EPS = 1e-5
G = B * P * K


_VT = 4096


def _vote1_body(x_ref, wo_ref, bo_ref, wf_ref, bf_ref, yo_ref, yf_ref, st_ref):
    i = pl.program_id(0)
    x = x_ref[...]
    yo = jnp.dot(x, wo_ref[...], preferred_element_type=jnp.float32) + bo_ref[...]
    yf = jnp.dot(x, wf_ref[...], preferred_element_type=jnp.float32) + bf_ref[...]
    yo_ref[...] = yo
    yf_ref[...] = yf
    z = jnp.zeros((4, CSEED), jnp.float32)
    s = jnp.concatenate([
        jnp.sum(yo, axis=0, keepdims=True),
        jnp.sum(yo * yo, axis=0, keepdims=True),
        jnp.sum(yf, axis=0, keepdims=True),
        jnp.sum(yf * yf, axis=0, keepdims=True),
        z,
    ], axis=0)

    @pl.when(i == 0)
    def _():
        st_ref[...] = s

    @pl.when(i > 0)
    def _():
        st_ref[...] = st_ref[...] + s


def _vote_pass1(x, wo_t, bo, wf_t, bf):
    nblk = T // _VT
    return pl.pallas_call(
        _vote1_body,
        grid=(nblk,),
        in_specs=[
            pl.BlockSpec((_VT, CSEED), lambda i: (i, 0)),
            pl.BlockSpec((CSEED, CSEED), lambda i: (0, 0)),
            pl.BlockSpec((1, CSEED), lambda i: (0, 0)),
            pl.BlockSpec((CSEED, CSEED), lambda i: (0, 0)),
            pl.BlockSpec((1, CSEED), lambda i: (0, 0)),
        ],
        out_specs=[
            pl.BlockSpec((_VT, CSEED), lambda i: (i, 0)),
            pl.BlockSpec((_VT, CSEED), lambda i: (i, 0)),
            pl.BlockSpec((8, CSEED), lambda i: (0, 0)),
        ],
        out_shape=[
            jax.ShapeDtypeStruct((T, CSEED), jnp.float32),
            jax.ShapeDtypeStruct((T, CSEED), jnp.float32),
            jax.ShapeDtypeStruct((8, CSEED), jnp.float32),
        ],
    )(x, wo_t, bo, wf_t, bf)


def _vote2_body(yo_ref, yf_ref, st_ref, sxyz_ref, go_ref, beo_ref, gf_ref,
                bef_ref, wo2_ref, bo2_ref, wf2_ref, bf2_ref, votes_ref, vf_ref):
    st = st_ref[...]
    n = jnp.float32(T)
    mo = st[0:1, :] / n
    vo = st[1:2, :] / n - mo * mo
    mf = st[2:3, :] / n
    vf_v = st[3:4, :] / n - mf * mf

    yo = yo_ref[...]
    xo = jax.nn.relu(go_ref[...] * (yo - mo) / jnp.sqrt(vo + EPS) + beo_ref[...])
    yf = yf_ref[...]
    xf = jax.nn.relu(gf_ref[...] * (yf - mf) / jnp.sqrt(vf_v + EPS) + bef_ref[...])

    off = jnp.dot(xo, wo2_ref[...], preferred_element_type=jnp.float32)
    votes_ref[...] = sxyz_ref[...] + (off[:, :2] + bo2_ref[0:1, :2])

    vf_ref[...] = jnp.dot(xf, wf2_ref[...], preferred_element_type=jnp.float32) + bf2_ref[...]


def _vote_pass2(yo, yf, st, sxyz, go, beo, gf, bef, wo2, bo2, wf2_t, bf2):
    nblk = T // _VT
    return pl.pallas_call(
        _vote2_body,
        grid=(nblk,),
        in_specs=[
            pl.BlockSpec((_VT, CSEED), lambda i: (i, 0)),
            pl.BlockSpec((_VT, CSEED), lambda i: (i, 0)),
            pl.BlockSpec((8, CSEED), lambda i: (0, 0)),
            pl.BlockSpec((_VT, 2), lambda i: (i, 0)),
            pl.BlockSpec((1, CSEED), lambda i: (0, 0)),
            pl.BlockSpec((1, CSEED), lambda i: (0, 0)),
            pl.BlockSpec((1, CSEED), lambda i: (0, 0)),
            pl.BlockSpec((1, CSEED), lambda i: (0, 0)),
            pl.BlockSpec((CSEED, CSEED), lambda i: (0, 0)),
            pl.BlockSpec((1, CSEED), lambda i: (0, 0)),
            pl.BlockSpec((CSEED, FVOTE), lambda i: (0, 0)),
            pl.BlockSpec((1, FVOTE), lambda i: (0, 0)),
        ],
        out_specs=[
            pl.BlockSpec((_VT, 2), lambda i: (i, 0)),
            pl.BlockSpec((_VT, FVOTE), lambda i: (i, 0)),
        ],
        out_shape=[
            jax.ShapeDtypeStruct((T, 2), jnp.float32),
            jax.ShapeDtypeStruct((T, FVOTE), jnp.float32),
        ],
    )(yo, yf, st, sxyz, go, beo, gf, bef, wo2, bo2, wf2_t, bf2)


def _fps_body(vx_ref, vy_ref, nx_ref, ny_ref, mind_ref, far_ref):
    i = pl.program_id(0)

    @pl.when(i == 0)
    def _():
        mind_ref[...] = jnp.full((B, M), 1e10, jnp.float32)
        far_ref[...] = jnp.zeros((8, 128), jnp.int32)

    far = far_ref[:, 0:1]
    lanes = lax.broadcasted_iota(jnp.int32, (B, M), 1)
    selm = lanes == far
    vx = vx_ref[...]
    vy = vy_ref[...]
    cx = jnp.sum(jnp.where(selm, vx, 0.0), axis=1, keepdims=True)
    cy = jnp.sum(jnp.where(selm, vy, 0.0), axis=1, keepdims=True)
    planes = lax.broadcasted_iota(jnp.int32, (B, P), 1)
    colm = planes == i

    @pl.when(i == 0)
    def _():
        nx_ref[...] = jnp.zeros((B, P), jnp.float32)
        ny_ref[...] = jnp.zeros((B, P), jnp.float32)

    nx_ref[...] = jnp.where(colm, cx, nx_ref[...])
    ny_ref[...] = jnp.where(colm, cy, ny_ref[...])

    dx = vx - cx
    dy = vy - cy
    d = dx * dx + dy * dy
    mind = jnp.minimum(mind_ref[...], d)
    mind_ref[...] = mind
    far_new = jnp.argmax(mind, axis=1).astype(jnp.int32)
    far_ref[:, 0:1] = far_new[:, None]


def _fps(vx, vy):
    return pl.pallas_call(
        _fps_body,
        grid=(P,),
        in_specs=[
            pl.BlockSpec((B, M), lambda i: (0, 0)),
            pl.BlockSpec((B, M), lambda i: (0, 0)),
        ],
        out_specs=[
            pl.BlockSpec((B, P), lambda i: (0, 0)),
            pl.BlockSpec((B, P), lambda i: (0, 0)),
        ],
        out_shape=[
            jax.ShapeDtypeStruct((B, P), jnp.float32),
            jax.ShapeDtypeStruct((B, P), jnp.float32),
        ],
        scratch_shapes=[
            pltpu.VMEM((B, M), jnp.float32),
            pltpu.VMEM((8, 128), jnp.int32),
        ],
    )(vx, vy)


_PR = 64
_NC = 256
_CW = 64

_INF = float('inf')


def _ballq_body(vx_ref, vy_ref, nxy_ref, out_ref, d_ref):
    b = pl.program_id(0)
    r = pl.program_id(1)
    row0 = b * P + r * _PR
    nx = nxy_ref[pl.ds(row0, _PR), 0:1].reshape(_PR, 1, 1)
    ny = nxy_ref[pl.ds(row0, _PR), 1:2].reshape(_PR, 1, 1)
    vx3 = vx_ref[pl.ds(b, 1), :, :]
    vy3 = vy_ref[pl.ds(b, 1), :, :]
    dx = nx - vx3
    dy = ny - vy3
    d3 = dx * dx + dy * dy
    d_ref[...] = d3

    wio3 = lax.broadcasted_iota(jnp.int32, (_PR, _CW, _NC), 1).astype(jnp.float32)
    chunkf = lax.broadcasted_iota(jnp.int32, (_PR, _NC), 1).astype(jnp.float32)
    lane2 = lax.broadcasted_iota(jnp.int32, (_PR, _NC), 1)

    NL = 6
    ms, ps = [], []
    dcur = d3
    for _ in range(NL):
        mi = jnp.min(dcur, axis=1)
        pi = jnp.min(jnp.where(dcur == mi[:, None, :], wio3, _INF), axis=1)
        ms.append(mi)
        ps.append(pi)
        dcur = jnp.where(wio3 == pi[:, None, :], _INF, dcur)

    cmin = ms[0]
    cpos = ps[0]
    cnt = jnp.zeros((_PR, _NC), jnp.int32)
    bad = jnp.zeros((_PR, _NC), jnp.bool_)

    cols = []
    for k in range(K):
        m = jnp.min(cmin, axis=1, keepdims=True)
        oidx = cpos * float(_NC) + chunkf
        osel = jnp.min(jnp.where(cmin == m, oidx, _INF), axis=1)
        osel_i = osel.astype(jnp.int32)
        c = osel_i & (_NC - 1)
        mv = m[:, 0]
        jeff = jnp.where(jnp.sqrt(mv) > RADIUS, 0, osel_i) + b * M
        cols.append(jeff[:, None])

        onehot = lane2 == c[:, None]
        cnt = cnt + onehot.astype(jnp.int32)
        nxt_v = jnp.full((_PR, _NC), _INF, jnp.float32)
        nxt_p = jnp.zeros((_PR, _NC), jnp.float32)
        for lvl in range(1, NL):
            is_l = cnt == lvl
            nxt_v = jnp.where(is_l, ms[lvl], nxt_v)
            nxt_p = jnp.where(is_l, ps[lvl], nxt_p)
        cmin = jnp.where(onehot, nxt_v, cmin)
        cpos = jnp.where(onehot, nxt_p, cpos)
        bad = bad | (onehot & (cnt >= NL))

    out_ref[0] = jnp.concatenate(cols, axis=1)
    anyb = jnp.any(bad)

    @pl.when(anyb)
    def _():
        oidx3 = (wio3 * float(_NC) +
                 lax.broadcasted_iota(jnp.int32, (_PR, _CW, _NC), 2)
                 .astype(jnp.float32))
        fcols = []
        for k in range(K):
            dd = d_ref[...]
            fm2 = jnp.min(dd, axis=1)
            fm = jnp.min(fm2, axis=1)
            sel = jnp.where(dd == fm[:, None, None], oidx3, _INF)
            fo2 = jnp.min(sel, axis=1)
            fo = jnp.min(fo2, axis=1)
            d_ref[...] = jnp.where(oidx3 == fo[:, None, None], _INF, dd)
            fo_i = fo.astype(jnp.int32)
            fj = jnp.where(jnp.sqrt(fm) > RADIUS, 0, fo_i) + b * M
            fcols.append(fj[:, None])
        out_ref[0] = jnp.concatenate(fcols, axis=1)


def _ballq(vx3, vy3, nxy):
    return pl.pallas_call(
        _ballq_body,
        grid=(B, P // _PR),
        in_specs=[
            pl.BlockSpec((B, _CW, _NC), lambda b, r: (0, 0, 0)),
            pl.BlockSpec((B, _CW, _NC), lambda b, r: (0, 0, 0)),
            pl.BlockSpec((B * P, 2), lambda b, r: (0, 0)),
        ],
        out_specs=pl.BlockSpec((1, _PR, K), lambda b, r: (b, r, 0)),
        out_shape=jax.ShapeDtypeStruct((B, P, K), jnp.int32),
        scratch_shapes=[
            pltpu.VMEM((_PR, _CW, _NC), jnp.float32),
        ],
    )(vx3, vy3, nxy)


_NW = 32
_RPW = G // _NW
_NCH = _RPW // 128


def _scgather_body(gidx_hbm, feats_hbm, vx_hbm, vy_hbm, nx_hbm, ny_hbm,
                   fout_hbm, xout_hbm, yout_hbm,
                   idx_v, rows_v, vx_v, vy_v, nx_v, ny_v, xg_v, yg_v, sem):
    wid = lax.axis_index("s") * 2 + lax.axis_index("c")
    base = wid * _RPW
    b = base // (P * K)
    pltpu.sync_copy(vx_hbm.at[pl.ds(b * M, M)], vx_v)
    pltpu.sync_copy(vy_hbm.at[pl.ds(b * M, M)], vy_v)
    pltpu.sync_copy(nx_hbm.at[pl.ds(b * P, P)], nx_v)
    pltpu.sync_copy(ny_hbm.at[pl.ds(b * P, P)], ny_v)
    for c in range(_NCH):
        row0 = base + c * 128
        pltpu.sync_copy(gidx_hbm.at[pl.ds(row0, 128)], idx_v)
        pltpu.async_copy(feats_hbm.at[idx_v], rows_v, sem).wait()
        pltpu.sync_copy(rows_v, fout_hbm.at[pl.ds(row0, 128)])
        for k in range(8):
            off = c * 128 + k * 16
            jv = idx_v[pl.ds(k * 16, 16)] - b * M
            rv = (base + off) + lax.iota(jnp.int32, 16)
            pv = (rv & (P * K - 1)) >> 5
            xc = plsc.load_gather(nx_v, [pv])
            yc = plsc.load_gather(ny_v, [pv])
            xv = plsc.load_gather(vx_v, [jv])
            yv = plsc.load_gather(vy_v, [jv])
            xg_v[pl.ds(off, 16)] = xv - xc
            yg_v[pl.ds(off, 16)] = yv - yc
    pltpu.sync_copy(xg_v, xout_hbm.at[pl.ds(base, _RPW)])
    pltpu.sync_copy(yg_v, yout_hbm.at[pl.ds(base, _RPW)])


def _scgather(gidx, feats, vx_flat, vy_flat, nx_flat, ny_flat):
    mesh = plsc.VectorSubcoreMesh(core_axis_name="c", subcore_axis_name="s")
    fn = functools.partial(
        pl.kernel,
        mesh=mesh,
        compiler_params=pltpu.CompilerParams(needs_layout_passes=False),
        out_type=[
            jax.ShapeDtypeStruct((G, FVOTE), jnp.float32),
            jax.ShapeDtypeStruct((G,), jnp.float32),
            jax.ShapeDtypeStruct((G,), jnp.float32),
        ],
        scratch_types=[
            pltpu.VMEM((128,), jnp.int32),
            pltpu.VMEM((128, FVOTE), jnp.float32),
            pltpu.VMEM((M,), jnp.float32),
            pltpu.VMEM((M,), jnp.float32),
            pltpu.VMEM((P,), jnp.float32),
            pltpu.VMEM((P,), jnp.float32),
            pltpu.VMEM((_RPW,), jnp.float32),
            pltpu.VMEM((_RPW,), jnp.float32),
            pltpu.SemaphoreType.DMA,
        ],
    )(_scgather_body)
    return fn(gidx, feats, vx_flat, vy_flat, nx_flat, ny_flat)


_GT = 2048


def _agg1_body(fg_ref, xgn_ref, ygn_ref, w1f_ref, c01_ref, b1_ref, y_ref, st_ref):
    i = pl.program_id(0)
    y = jnp.dot(fg_ref[...], w1f_ref[...], preferred_element_type=jnp.float32)
    y = y + xgn_ref[...] * c01_ref[0:1, :] + ygn_ref[...] * c01_ref[1:2, :] + b1_ref[...]
    y_ref[...] = y
    z = jnp.zeros((6, AGG), jnp.float32)
    s = jnp.concatenate([
        jnp.sum(y, axis=0, keepdims=True),
        jnp.sum(y * y, axis=0, keepdims=True),
        z,
    ], axis=0)

    @pl.when(i == 0)
    def _():
        st_ref[...] = s

    @pl.when(i > 0)
    def _():
        st_ref[...] = st_ref[...] + s


def _agg1(fg, xgn, ygn, w1f_t, c01, b1):
    nblk = G // _GT
    return pl.pallas_call(
        _agg1_body,
        grid=(nblk,),
        in_specs=[
            pl.BlockSpec((_GT, FVOTE), lambda i: (i, 0)),
            pl.BlockSpec((_GT, 1), lambda i: (i, 0)),
            pl.BlockSpec((_GT, 1), lambda i: (i, 0)),
            pl.BlockSpec((FVOTE, AGG), lambda i: (0, 0)),
            pl.BlockSpec((8, AGG), lambda i: (0, 0)),
            pl.BlockSpec((1, AGG), lambda i: (0, 0)),
        ],
        out_specs=[
            pl.BlockSpec((_GT, AGG), lambda i: (i, 0)),
            pl.BlockSpec((8, AGG), lambda i: (0, 0)),
        ],
        out_shape=[
            jax.ShapeDtypeStruct((G, AGG), jnp.float32),
            jax.ShapeDtypeStruct((8, AGG), jnp.float32),
        ],
    )(fg, xgn, ygn, w1f_t, c01, b1)


def _agg2_body(y1_ref, st_ref, g_ref, be_ref, w2_ref, b2_ref, y_ref, st2_ref):
    i = pl.program_id(0)
    st = st_ref[...]
    n = jnp.float32(G)
    m = st[0:1, :] / n
    v = st[1:2, :] / n - m * m
    h = jax.nn.relu(g_ref[...] * (y1_ref[...] - m) / jnp.sqrt(v + EPS) + be_ref[...])
    y = jnp.dot(h, w2_ref[...], preferred_element_type=jnp.float32) + b2_ref[...]
    y_ref[...] = y
    z = jnp.zeros((6, AGG), jnp.float32)
    s = jnp.concatenate([
        jnp.sum(y, axis=0, keepdims=True),
        jnp.sum(y * y, axis=0, keepdims=True),
        z,
    ], axis=0)

    @pl.when(i == 0)
    def _():
        st2_ref[...] = s

    @pl.when(i > 0)
    def _():
        st2_ref[...] = st2_ref[...] + s


def _agg2(y1, st1, g, be, w2_t, b2):
    nblk = G // _GT
    return pl.pallas_call(
        _agg2_body,
        grid=(nblk,),
        in_specs=[
            pl.BlockSpec((_GT, AGG), lambda i: (i, 0)),
            pl.BlockSpec((8, AGG), lambda i: (0, 0)),
            pl.BlockSpec((1, AGG), lambda i: (0, 0)),
            pl.BlockSpec((1, AGG), lambda i: (0, 0)),
            pl.BlockSpec((AGG, AGG), lambda i: (0, 0)),
            pl.BlockSpec((1, AGG), lambda i: (0, 0)),
        ],
        out_specs=[
            pl.BlockSpec((_GT, AGG), lambda i: (i, 0)),
            pl.BlockSpec((8, AGG), lambda i: (0, 0)),
        ],
        out_shape=[
            jax.ShapeDtypeStruct((G, AGG), jnp.float32),
            jax.ShapeDtypeStruct((8, AGG), jnp.float32),
        ],
    )(y1, st1, g, be, w2_t, b2)


def _agg3_body(y2_ref, st_ref, g_ref, be_ref, pool_ref):
    st = st_ref[...]
    n = jnp.float32(G)
    m = st[0:1, :] / n
    v = st[1:2, :] / n - m * m
    h = jax.nn.relu(g_ref[...] * (y2_ref[...] - m) / jnp.sqrt(v + EPS) + be_ref[...])
    h3 = h.reshape(_GT // K, K, AGG)
    pool_ref[...] = jnp.max(h3, axis=1)


def _agg3(y2, st2, g, be):
    nblk = G // _GT
    return pl.pallas_call(
        _agg3_body,
        grid=(nblk,),
        in_specs=[
            pl.BlockSpec((_GT, AGG), lambda i: (i, 0)),
            pl.BlockSpec((8, AGG), lambda i: (0, 0)),
            pl.BlockSpec((1, AGG), lambda i: (0, 0)),
            pl.BlockSpec((1, AGG), lambda i: (0, 0)),
        ],
        out_specs=pl.BlockSpec((_GT // K, AGG), lambda i: (i, 0)),
        out_shape=jax.ShapeDtypeStruct((B * P, AGG), jnp.float32),
    )(y2, st2, g, be)


_HT = B * P


def _head_body(x_ref, w1_ref, b1_ref, g1_ref, be1_ref, w2_ref, b2_ref, g2_ref,
               be2_ref, ow_ref, ob_ref, cw_ref, cb_ref, obj_ref, cls_ref):
    n = jnp.float32(_HT)
    y1 = jnp.dot(x_ref[...], w1_ref[...], preferred_element_type=jnp.float32) + b1_ref[...]
    m1 = jnp.sum(y1, axis=0, keepdims=True) / n
    v1 = jnp.sum(y1 * y1, axis=0, keepdims=True) / n - m1 * m1
    h1 = jax.nn.relu(g1_ref[...] * (y1 - m1) / jnp.sqrt(v1 + EPS) + be1_ref[...])
    y2 = jnp.dot(h1, w2_ref[...], preferred_element_type=jnp.float32) + b2_ref[...]
    m2 = jnp.sum(y2, axis=0, keepdims=True) / n
    v2 = jnp.sum(y2 * y2, axis=0, keepdims=True) / n - m2 * m2
    h2 = jax.nn.relu(g2_ref[...] * (y2 - m2) / jnp.sqrt(v2 + EPS) + be2_ref[...])
    obj_ref[...] = jnp.sum(h2 * ow_ref[...], axis=1, keepdims=True) + ob_ref[0:1, 0:1]
    cls_ref[...] = jnp.dot(h2, cw_ref[...], preferred_element_type=jnp.float32) + cb_ref[...]


def _head(x, w1_t, b1, g1, be1, w2_t, b2, g2, be2, ow, ob, cw_t, cb):
    return pl.pallas_call(
        _head_body,
        grid=(1,),
        in_specs=[
            pl.BlockSpec((_HT, AGG), lambda i: (0, 0)),
            pl.BlockSpec((AGG, AGG), lambda i: (0, 0)),
            pl.BlockSpec((1, AGG), lambda i: (0, 0)),
            pl.BlockSpec((1, AGG), lambda i: (0, 0)),
            pl.BlockSpec((1, AGG), lambda i: (0, 0)),
            pl.BlockSpec((AGG, AGG // 2), lambda i: (0, 0)),
            pl.BlockSpec((1, AGG // 2), lambda i: (0, 0)),
            pl.BlockSpec((1, AGG // 2), lambda i: (0, 0)),
            pl.BlockSpec((1, AGG // 2), lambda i: (0, 0)),
            pl.BlockSpec((1, AGG // 2), lambda i: (0, 0)),
            pl.BlockSpec((8, 8), lambda i: (0, 0)),
            pl.BlockSpec((AGG // 2, 16), lambda i: (0, 0)),
            pl.BlockSpec((1, 16), lambda i: (0, 0)),
        ],
        out_specs=[
            pl.BlockSpec((_HT, 1), lambda i: (0, 0)),
            pl.BlockSpec((_HT, 16), lambda i: (0, 0)),
        ],
        out_shape=[
            jax.ShapeDtypeStruct((_HT, 1), jnp.float32),
            jax.ShapeDtypeStruct((_HT, 16), jnp.float32),
        ],
    )(x, w1_t, b1, g1, be1, w2_t, b2, g2, be2, ow, ob, cw_t, cb)


def kernel(seed_xyz, seed_features, v_ow1, v_ob1, v_og1, v_obt1, v_ow2, v_ob2,
           v_fw1, v_fb1, v_fg1, v_fbt1, v_fw2, v_fb2, a_w1, a_b1, a_g1, a_bt1,
           a_w2, a_b2, a_g2, a_bt2, h_w1, h_b1, h_g1, h_bt1, h_w2, h_b2, h_g2,
           h_bt2, h_ow, h_ob, h_cw, h_cb):
    r1 = lambda a: a.reshape(1, -1)
    x = seed_features.reshape(T, CSEED)
    sxyz = seed_xyz.reshape(T, 2)

    yo, yf, st = _vote_pass1(x, v_ow1.T, r1(v_ob1), v_fw1.T, r1(v_fb1))
    wo2 = jnp.zeros((CSEED, CSEED), jnp.float32).at[:, :2].set(v_ow2.T)
    bo2 = jnp.zeros((1, CSEED), jnp.float32).at[0, :2].set(v_ob2)
    votes, vfeat = _vote_pass2(yo, yf, st, sxyz, r1(v_og1), r1(v_obt1),
                               r1(v_fg1), r1(v_fbt1), wo2, bo2, v_fw2.T,
                               r1(v_fb2))

    vx = votes[:, 0].reshape(B, M)
    vy = votes[:, 1].reshape(B, M)
    nx, ny = _fps(vx, vy)
    nxy = jnp.stack([nx.reshape(-1), ny.reshape(-1)], axis=1)

    gidx = _ballq(vx.reshape(B, _CW, _NC), vy.reshape(B, _CW, _NC),
                  nxy).reshape(G)

    fg, xgn, ygn = _scgather(gidx, vfeat, votes[:, 0], votes[:, 1],
                             nx.reshape(-1), ny.reshape(-1))

    c01 = jnp.zeros((8, AGG), jnp.float32).at[:2].set(a_w1[:, :2].T)
    y1, st1 = _agg1(fg, xgn.reshape(G, 1), ygn.reshape(G, 1), a_w1[:, 2:].T,
                    c01, r1(a_b1))
    y2, st2 = _agg2(y1, st1, r1(a_g1), r1(a_bt1), a_w2.T, r1(a_b2))
    pooled = _agg3(y2, st2, r1(a_g2), r1(a_bt2))

    ow = jnp.zeros((8, 8), jnp.float32).at[0, 0].set(h_ob[0])
    cw = jnp.zeros((16, AGG // 2), jnp.float32).at[:NCLS].set(h_cw)
    cb = jnp.zeros((1, 16), jnp.float32).at[0, :NCLS].set(h_cb)
    obj2, cls2 = _head(pooled, h_w1.T, r1(h_b1), r1(h_g1), r1(h_bt1), h_w2.T,
                       r1(h_b2), r1(h_g2), r1(h_bt2), r1(h_ow), ow, cw.T, cb)

    obj = obj2[:, 0].reshape(B, P)
    cls = cls2[:, :NCLS].reshape(B, P, NCLS)
    new_xyz = jnp.stack([nx, ny], axis=-1)
    return (obj, cls, new_xyz)
